# baseline (speedup 1.0000x reference)
import functools

import jax
import jax.numpy as jnp
from jax import lax
from jax.experimental import pallas as pl
from jax.experimental.pallas import tpu as pltpu
from jax.experimental.pallas import tpu_sc as plsc

_R = 128


@functools.lru_cache(maxsize=None)
def _make_gather(V, D, B):
    info = plsc.get_sparse_core_info()
    NC, NS = info.num_cores, info.num_subcores
    NW = NC * NS
    assert B % (NW * _R) == 0
    bpw = B // NW
    K = 8
    C = K * _R
    assert bpw % C == 0
    steps = bpw // C

    mesh = plsc.VectorSubcoreMesh(core_axis_name="c", subcore_axis_name="s")

    @functools.partial(
        pl.kernel,
        mesh=mesh,
        compiler_params=pltpu.CompilerParams(use_tc_tiling_on_sc=False),
        out_type=jax.ShapeDtypeStruct((B, D), jnp.float32),
        scratch_types=[
            pltpu.VMEM((K, _R), jnp.int32),
            pltpu.VMEM((C, D), jnp.float32),
            pltpu.SemaphoreType.DMA,
        ],
    )
    def gather_kernel(table_hbm, idx_hbm, out_hbm, idx_v, rows_v, sem):
        wid = lax.axis_index("s") * NC + lax.axis_index("c")
        row0 = wid * (bpw // _R)
        base = wid * bpw

        def step(g, carry):
            pltpu.sync_copy(idx_hbm.at[pl.ds(row0 + g * K, K)], idx_v)
            cps = [
                pltpu.make_async_copy(
                    table_hbm.at[idx_v.at[j]],
                    rows_v.at[pl.ds(j * _R, _R)],
                    sem,
                )
                for j in range(K)
            ]
            for c in cps:
                c.start()
            for c in cps:
                c.wait()
            pltpu.sync_copy(rows_v, out_hbm.at[pl.ds(base + g * C, C)])
            return carry

        lax.fori_loop(0, steps, step, 0)

    return gather_kernel


def kernel(indices, table):
    Bt, H = indices.shape
    V, D = table.shape
    B = Bt * H
    idx2d = indices.reshape(B // _R, _R)
    out = _make_gather(V, D, B)(table, idx2d)
    return out.reshape(Bt, H, D)


# speedup vs baseline: 1.1930x; 1.1930x over previous
import functools

import jax
import jax.numpy as jnp
from jax import lax
from jax.experimental import pallas as pl
from jax.experimental.pallas import tpu as pltpu
from jax.experimental.pallas import tpu_sc as plsc

_R = 128
_K = 8
_NB = 2


@functools.lru_cache(maxsize=None)
def _make_gather(V, D, B):
    info = plsc.get_sparse_core_info()
    NC, NS = info.num_cores, info.num_subcores
    NW = NC * NS
    C = _K * _R
    assert B % (NW * C) == 0
    bpw = B // NW
    steps = bpw // C
    krows = bpw // _R

    mesh = plsc.VectorSubcoreMesh(core_axis_name="c", subcore_axis_name="s")

    @functools.partial(
        pl.kernel,
        mesh=mesh,
        compiler_params=pltpu.CompilerParams(use_tc_tiling_on_sc=False),
        out_type=jax.ShapeDtypeStruct((B // _R, _R, D), jnp.float32),
        scratch_types=[
            pltpu.VMEM((krows, _R), jnp.int32),
            pltpu.VMEM((_NB, _K, _R, D), jnp.float32),
            pltpu.SemaphoreType.DMA((_NB,)),
            pltpu.SemaphoreType.DMA((_NB,)),
        ],
    )
    def gather_kernel(table_hbm, idx_hbm, out_hbm, idx_v, rows_v, gsem, osem):
        wid = lax.axis_index("s") * NC + lax.axis_index("c")
        row0 = wid * krows

        pltpu.sync_copy(idx_hbm.at[pl.ds(row0, krows)], idx_v)

        def gather_descs(g, b):
            return [
                pltpu.make_async_copy(
                    table_hbm.at[idx_v.at[g * _K + j]],
                    rows_v.at[b, j],
                    gsem.at[b],
                )
                for j in range(_K)
            ]

        def out_desc(g, b):
            return pltpu.make_async_copy(
                rows_v.at[b], out_hbm.at[pl.ds(row0 + g * _K, _K)], osem.at[b]
            )

        for c in gather_descs(0, 0):
            c.start()

        def step(g, carry):
            b = lax.rem(g, _NB)
            nb = lax.rem(g + 1, _NB)

            @pl.when(g >= 1)
            def _():
                out_desc(g - 1, nb).wait()

            @pl.when(g + 1 < steps)
            def _():
                for c in gather_descs(g + 1, nb):
                    c.start()

            for c in gather_descs(g, b):
                c.wait()
            out_desc(g, b).start()
            return carry

        lax.fori_loop(0, steps, step, 0)
        out_desc(steps - 1, lax.rem(steps - 1, _NB)).wait()

    return gather_kernel


def kernel(indices, table):
    Bt, H = indices.shape
    V, D = table.shape
    B = Bt * H
    idx2d = indices.reshape(B // _R, _R)
    out = _make_gather(V, D, B)(table, idx2d)
    return out.reshape(Bt, H, D)


# speedup vs baseline: 1.1936x; 1.0006x over previous
import functools

import jax
import jax.numpy as jnp
from jax import lax
from jax.experimental import pallas as pl
from jax.experimental.pallas import tpu as pltpu
from jax.experimental.pallas import tpu_sc as plsc

_R = 128
_K = 8
_NB = 4


@functools.lru_cache(maxsize=None)
def _make_gather(V, D, B):
    info = plsc.get_sparse_core_info()
    NC, NS = info.num_cores, info.num_subcores
    NW = NC * NS
    C = _K * _R
    assert B % (NW * C) == 0
    bpw = B // NW
    steps = bpw // C
    krows = bpw // _R

    mesh = plsc.VectorSubcoreMesh(core_axis_name="c", subcore_axis_name="s")

    @functools.partial(
        pl.kernel,
        mesh=mesh,
        compiler_params=pltpu.CompilerParams(use_tc_tiling_on_sc=False),
        out_type=jax.ShapeDtypeStruct((B // _R, _R, D), jnp.float32),
        scratch_types=[
            pltpu.VMEM((krows, _R), jnp.int32),
            pltpu.VMEM((_NB, _K, _R, D), jnp.float32),
            pltpu.SemaphoreType.DMA((_NB,)),
            pltpu.SemaphoreType.DMA((_NB,)),
        ],
    )
    def gather_kernel(table_hbm, idx_hbm, out_hbm, idx_v, rows_v, gsem, osem):
        wid = lax.axis_index("s") * NC + lax.axis_index("c")
        row0 = wid * krows

        pltpu.sync_copy(idx_hbm.at[pl.ds(row0, krows)], idx_v)

        def gather_descs(g, b):
            return [
                pltpu.make_async_copy(
                    table_hbm.at[idx_v.at[g * _K + j]],
                    rows_v.at[b, j],
                    gsem.at[b],
                )
                for j in range(_K)
            ]

        def out_desc(g, b):
            return pltpu.make_async_copy(
                rows_v.at[b], out_hbm.at[pl.ds(row0 + g * _K, _K)], osem.at[b]
            )

        for t in range(_NB - 1):
            for c in gather_descs(t, t):
                c.start()

        def step(g, carry):
            b = lax.rem(g, _NB)
            fb = lax.rem(g + _NB - 1, _NB)

            @pl.when(g >= 1)
            def _():
                out_desc(g - 1, fb).wait()

            @pl.when(g + _NB - 1 < steps)
            def _():
                for c in gather_descs(g + _NB - 1, fb):
                    c.start()

            for c in gather_descs(g, b):
                c.wait()
            out_desc(g, b).start()
            return carry

        lax.fori_loop(0, steps, step, 0)
        out_desc(steps - 1, lax.rem(steps - 1, _NB)).wait()

    return gather_kernel


def kernel(indices, table):
    Bt, H = indices.shape
    V, D = table.shape
    B = Bt * H
    idx2d = indices.reshape(B // _R, _R)
    out = _make_gather(V, D, B)(table, idx2d)
    return out.reshape(Bt, H, D)
